# baseline (speedup 1.0000x reference)
import functools

import jax
import jax.numpy as jnp
from jax import lax
from jax.experimental import pallas as pl
from jax.experimental.pallas import tpu as pltpu
from jax.experimental.pallas import tpu_sc as plsc

NUM_POSES = 1000
EMBED_DIM = 128
BATCH = 16384

NC = 2
NS = 16
NW = NC * NS
B_PER_W = BATCH // NW
CHUNK = 128
NCHUNK = B_PER_W // CHUNK
STAGE_TILES = 5
STAGE_ROWS = NUM_POSES // STAGE_TILES


def _make_gather():
    mesh = plsc.VectorSubcoreMesh(core_axis_name="c", subcore_axis_name="s")

    @functools.partial(
        pl.kernel,
        mesh=mesh,
        out_type=jax.ShapeDtypeStruct((BATCH, EMBED_DIM), jnp.float32),
        scratch_types=[
            pltpu.VMEM_SHARED((NUM_POSES, EMBED_DIM), jnp.float32),
            pltpu.VMEM((B_PER_W,), jnp.int32),
            [pltpu.VMEM((CHUNK, EMBED_DIM), jnp.float32)
             for _ in range(NCHUNK)],
            pltpu.SemaphoreType.DMA,
            pltpu.SemaphoreType.DMA,
            pltpu.SemaphoreType.DMA,
            pltpu.SemaphoreType.DMA,
        ],
    )
    def gather_kernel(x_hbm, table_hbm, out_hbm, tab_s, idx_v, bufs,
                      isem, hsem, ssem, wsem):
        sid = lax.axis_index("s")
        wid = sid * NC + lax.axis_index("c")
        base = wid * B_PER_W

        idx_cp = pltpu.async_copy(x_hbm.at[pl.ds(base, B_PER_W)], idx_v, isem)

        @pl.when(sid < STAGE_TILES)
        def _stage():
            r0 = sid * STAGE_ROWS
            pltpu.sync_copy(table_hbm.at[pl.ds(r0, STAGE_ROWS)],
                            tab_s.at[pl.ds(r0, STAGE_ROWS)])

        idx_cp.wait()
        gathers = [None] * NCHUNK
        for c in range(1, NCHUNK, 2):
            gathers[c] = pltpu.async_copy(
                table_hbm.at[idx_v.at[pl.ds(c * CHUNK, CHUNK)]],
                bufs[c], hsem)
        plsc.subcore_barrier()
        for c in range(0, NCHUNK, 2):
            gathers[c] = pltpu.async_copy(
                tab_s.at[idx_v.at[pl.ds(c * CHUNK, CHUNK)]],
                bufs[c], ssem)
        writes = []
        for c in range(NCHUNK):
            gathers[c].wait()
            writes.append(pltpu.async_copy(
                bufs[c], out_hbm.at[pl.ds(base + c * CHUNK, CHUNK)], wsem))
        for w in writes:
            w.wait()

    return gather_kernel


_gather = jax.jit(_make_gather())


def kernel(x, table):
    return _gather(x, table)


# speedup vs baseline: 1.1039x; 1.1039x over previous
import functools

import jax
import jax.numpy as jnp
from jax import lax
from jax.experimental import pallas as pl
from jax.experimental.pallas import tpu as pltpu
from jax.experimental.pallas import tpu_sc as plsc

NUM_POSES = 1000
EMBED_DIM = 128
BATCH = 16384

NC = 2
NS = 16
NW = NC * NS
B_PER_W = BATCH // NW
CHUNK = 128
NCHUNK = B_PER_W // CHUNK
STAGE_ROWS = 64
LAST_STAGE_ROW = NUM_POSES - STAGE_ROWS


def _make_gather():
    mesh = plsc.VectorSubcoreMesh(core_axis_name="c", subcore_axis_name="s")

    @functools.partial(
        pl.kernel,
        mesh=mesh,
        out_type=jax.ShapeDtypeStruct((BATCH, EMBED_DIM), jnp.float32),
        scratch_types=[
            pltpu.VMEM_SHARED((NUM_POSES, EMBED_DIM), jnp.float32),
            pltpu.VMEM((B_PER_W,), jnp.int32),
            [pltpu.VMEM((CHUNK, EMBED_DIM), jnp.float32)
             for _ in range(NCHUNK)],
            pltpu.SemaphoreType.DMA,
            pltpu.SemaphoreType.DMA,
            pltpu.SemaphoreType.DMA,
        ],
    )
    def gather_kernel(x_hbm, table_hbm, out_hbm, tab_s, idx_v, bufs,
                      isem, gsem, wsem):
        sid = lax.axis_index("s")
        wid = sid * NC + lax.axis_index("c")
        base = wid * B_PER_W

        idx_cp = pltpu.async_copy(x_hbm.at[pl.ds(base, B_PER_W)], idx_v, isem)

        r0 = pl.multiple_of(jnp.minimum(sid * STAGE_ROWS, LAST_STAGE_ROW), 8)
        pltpu.sync_copy(table_hbm.at[pl.ds(r0, STAGE_ROWS)],
                        tab_s.at[pl.ds(r0, STAGE_ROWS)])

        idx_cp.wait()
        plsc.subcore_barrier()
        gathers = []
        for c in range(NCHUNK):
            gathers.append(pltpu.async_copy(
                tab_s.at[idx_v.at[pl.ds(c * CHUNK, CHUNK)]],
                bufs[c], gsem))
        writes = []
        for c in range(NCHUNK):
            gathers[c].wait()
            writes.append(pltpu.async_copy(
                bufs[c], out_hbm.at[pl.ds(base + c * CHUNK, CHUNK)], wsem))
        for w in writes:
            w.wait()

    return gather_kernel


_gather = jax.jit(_make_gather())


def kernel(x, table):
    return _gather(x, table)
